# baseline (speedup 1.0000x reference)
import functools

import jax
import jax.numpy as jnp
from jax import lax
from jax.experimental import pallas as pl
from jax.experimental.pallas import tpu as pltpu
from jax.experimental.pallas import tpu_sc as plsc

_NUM_CORES = 2
_NUM_SUBCORES = 16
_NUM_WORKERS = _NUM_CORES * _NUM_SUBCORES
_LANES = 16


@functools.lru_cache(maxsize=None)
def _build(B, PL_LEN, V, D):
    assert B % (_NUM_WORKERS * _LANES) == 0
    bpw = B // _NUM_WORKERS
    nblk = bpw // _LANES
    assert D == _LANES

    mesh = plsc.VectorSubcoreMesh(
        core_axis_name="c", subcore_axis_name="s",
        num_cores=_NUM_CORES, num_subcores=_NUM_SUBCORES)

    @functools.partial(
        pl.kernel,
        out_type=jax.ShapeDtypeStruct((B * D,), jnp.float32),
        mesh=mesh,
        compiler_params=pltpu.CompilerParams(needs_layout_passes=False),
        scratch_types=[
            pltpu.VMEM((bpw * PL_LEN,), jnp.int32),
            pltpu.VMEM((V * D * _LANES,), jnp.float32),
            pltpu.VMEM((bpw * (D + 1),), jnp.float32),
            pltpu.VMEM((bpw * D,), jnp.float32),
        ],
    )
    def plate_encode(idx_hbm, table_hbm, out_hbm, idx_v, table_v, pad_v, out_v):
        wid = lax.axis_index("s") * _NUM_CORES + lax.axis_index("c")
        base = wid * bpw
        with jax.named_scope("dma_in"):
            pltpu.sync_copy(idx_hbm.at[pl.ds(base * PL_LEN, bpw * PL_LEN)], idx_v)
            pltpu.sync_copy(table_hbm, table_v)

        lane = lax.broadcasted_iota(jnp.int32, (_LANES,), 0)
        lane_p = lane * PL_LEN
        lane_pad = lane * (D + 1)
        scale = jnp.float32(1.0 / PL_LEN)

        def block(b, carry):
            ivs = [plsc.load_gather(idx_v, [lane_p + (b * (_LANES * PL_LEN) + c)])
                   for c in range(PL_LEN)]
            base = [iv * (D * _LANES) + lane for iv in ivs]
            for d in range(D):
                acc = plsc.load_gather(table_v, [base[0] + (d * _LANES)])
                for c in range(1, PL_LEN):
                    acc = acc + plsc.load_gather(table_v, [base[c] + (d * _LANES)])
                plsc.store_scatter(pad_v,
                                   [lane_pad + (b * (_LANES * (D + 1)) + d)],
                                   acc * scale)
            for j in range(_LANES):
                p = b * _LANES + j
                out_v[pl.ds(p * D, D)] = pad_v[pl.ds(p * (D + 1), D)]
            return carry

        with jax.named_scope("compute"):
            lax.fori_loop(0, nblk, block, 0, unroll=False)
        with jax.named_scope("dma_out"):
            pltpu.sync_copy(out_v, out_hbm.at[pl.ds(base * D, bpw * D)])

    return plate_encode


def kernel(plates_indices, embedding_table):
    B, PL_LEN = plates_indices.shape
    V, D = embedding_table.shape
    fn = _build(B, PL_LEN, V, D)
    table_rep = jnp.broadcast_to(
        embedding_table.astype(jnp.float32)[:, :, None],
        (V, D, _LANES)).reshape(V * D * _LANES)
    out_flat = fn(plates_indices.astype(jnp.int32).reshape(B * PL_LEN),
                  table_rep)
    return out_flat.reshape(B, D)


# speedup vs baseline: 1.8082x; 1.8082x over previous
import functools

import jax
import jax.numpy as jnp
from jax import lax
from jax.experimental import pallas as pl
from jax.experimental.pallas import tpu as pltpu
from jax.experimental.pallas import tpu_sc as plsc

_NUM_CORES = 2
_NUM_SUBCORES = 16
_NUM_WORKERS = _NUM_CORES * _NUM_SUBCORES
_LANES = 16
_TSUB = 8
_TMIN = 128


@functools.lru_cache(maxsize=None)
def _build(B, PL_LEN, V, D):
    bpw = B // _NUM_WORKERS
    nblk = bpw // _LANES
    tpw = bpw // _TMIN
    drows = D // _TSUB
    nct = B // _TMIN
    assert bpw % _TMIN == 0 and D % _TSUB == 0 and PL_LEN == _TSUB
    assert D == _LANES

    mesh = plsc.VectorSubcoreMesh(
        core_axis_name="c", subcore_axis_name="s",
        num_cores=_NUM_CORES, num_subcores=_NUM_SUBCORES)

    @functools.partial(
        pl.kernel,
        out_type=jax.ShapeDtypeStruct((B * D,), jnp.float32),
        mesh=mesh,
        compiler_params=pltpu.CompilerParams(needs_layout_passes=False),
        scratch_types=[
            pltpu.VMEM((bpw * PL_LEN,), jnp.int32),
            pltpu.VMEM((V * D * _LANES,), jnp.float32),
            pltpu.VMEM((bpw * D,), jnp.float32),
        ],
    )
    def plate_encode(idx_hbm, table_hbm, out_hbm, idx_v, table_v, out_v):
        wid = lax.axis_index("s") * _NUM_CORES + lax.axis_index("c")
        pltpu.sync_copy(idx_hbm.at[pl.ds(wid * (bpw * PL_LEN), bpw * PL_LEN)],
                        idx_v)
        pltpu.sync_copy(table_hbm, table_v)

        lane = lax.broadcasted_iota(jnp.int32, (_LANES,), 0)
        scale = jnp.float32(1.0 / PL_LEN)
        tile_words = _TSUB * _TMIN

        def block(b, carry):
            boff = (b // (_TMIN // _LANES)) * tile_words \
                + (b % (_TMIN // _LANES)) * _LANES
            ivs = [idx_v[pl.ds(boff + c * _TMIN, _LANES)]
                   for c in range(PL_LEN)]
            bases = [iv * (D * _LANES) + lane for iv in ivs]
            for d in range(D):
                acc = plsc.load_gather(table_v, [bases[0] + (d * _LANES)])
                for c in range(1, PL_LEN):
                    acc = acc + plsc.load_gather(table_v,
                                                 [bases[c] + (d * _LANES)])
                ooff = (d // _TSUB) * (tpw * tile_words) \
                    + (b // (_TMIN // _LANES)) * tile_words \
                    + (d % _TSUB) * _TMIN \
                    + (b % (_TMIN // _LANES)) * _LANES
                out_v[pl.ds(ooff, _LANES)] = acc * scale
            return carry

        lax.fori_loop(0, nblk, block, 0, unroll=False)
        for r in range(drows):
            pltpu.sync_copy(
                out_v.at[pl.ds(r * (tpw * tile_words), tpw * tile_words)],
                out_hbm.at[pl.ds((r * nct + wid * tpw) * tile_words,
                                 tpw * tile_words)])

    return plate_encode


def kernel(plates_indices, embedding_table):
    B, PL_LEN = plates_indices.shape
    V, D = embedding_table.shape
    fn = _build(B, PL_LEN, V, D)
    idx_flat = (plates_indices.astype(jnp.int32)
                .reshape(B // _TMIN, _TMIN, PL_LEN)
                .transpose(0, 2, 1)
                .reshape(B * PL_LEN))
    table_rep = jnp.broadcast_to(
        embedding_table.astype(jnp.float32)[:, :, None],
        (V, D, _LANES)).reshape(V * D * _LANES)
    flat = fn(idx_flat, table_rep)
    return (flat.reshape(D // _TSUB, B // _TMIN, _TSUB, _TMIN)
            .transpose(1, 3, 0, 2)
            .reshape(B, D))
